# baseline (speedup 1.0000x reference)
import jax
import jax.numpy as jnp
from jax.experimental import pallas as pl
from jax.experimental.pallas import tpu as pltpu

_VMEM_LIMIT = 48 << 20
_B = 4


def _net_kernel(xp_ref, w0, g64, g128, wsc2, g256, wsc3, w42, wsc4, wfc, out_ref,
                padA, padP, padB, padBs, padC, padCs, padD, padD3, padE,
                wbd0, wbdA, wbd21, wbd22, wscb2):
    f32 = jnp.float32
    bf16 = jnp.bfloat16

    for p in (padA, padB, padBs, padC, padCs, padD, padD3, padE):
        p[...] = jnp.zeros(p.shape, p.dtype)
    for p in (wbd0, wbdA, wbd21, wbd22, wscb2):
        p[...] = jnp.zeros(p.shape, p.dtype)

    def fill_bd(scr, w, ci, co):
        scr[:, 0:ci, 0:co] = w
        scr[:, ci:2 * ci, co:2 * co] = w

    def conv3(pad, x, wslice, H, C, Co, B, extra=None, relu=True, lead=()):
        M = B * H * H
        acc = jnp.zeros((M, Co), f32)
        if x is not None:
            pad[:, 1:H + 1, :, :] = x
            zc = jnp.zeros((B, H + 2, 1, C), pad.dtype)
            full = pad[...]
            cols = [jnp.concatenate([zc, full[:, :, 0:H - 1, :]], axis=2),
                    full,
                    jnp.concatenate([full[:, :, 1:H, :], zc], axis=2)]
        else:
            cols = [pad[lead + (slice(None), slice(None),
                                slice(dj, dj + H), slice(None))]
                    for dj in range(3)]
        for dj in range(3):
            vdj = cols[dj]
            for di in range(3):
                xs = vdj[:, di:di + H, :, :].reshape(M, C)
                acc = acc + jnp.dot(xs, wslice(di * 3 + dj),
                                    preferred_element_type=f32)
        if extra is not None:
            acc = acc + extra
        if relu:
            acc = jnp.maximum(acc, 0.0)
        return acc.astype(bf16)

    def conv_s2(pad, x, wslice, H, C, Co, B):
        Ho = H // 2
        pad[:, 1:H + 1, 1:H + 1, :] = x.astype(f32)
        acc = jnp.zeros((B * Ho * Ho, Co), f32)
        for di in range(3):
            for dj in range(3):
                xs = pad[:, di:di + H:2, dj:dj + H:2, :].reshape(
                    B * Ho * Ho, C).astype(bf16)
                acc = acc + jnp.dot(xs, wslice(di * 3 + dj),
                                    preferred_element_type=f32)
        return acc

    fill_bd(wbd0, w0[...], 3, 64)
    a = conv3(xp_ref, None, lambda t: wbd0[t, 0:6, :], 32, 6, 128, 2, lead=(0,))

    fill_bd(wbdA, g64[:, :, 0:64], 64, 64)
    a = conv3(padA, a.reshape(2, 32, 32, 128), lambda t: wbdA[t], 32, 128, 128, 2)
    fill_bd(wbdA, g64[:, :, 64:128], 64, 64)
    a = conv3(padA, a.reshape(2, 32, 32, 128), lambda t: wbdA[t], 32, 128, 128, 2)

    padP[...] = a.reshape(2, 32, 32, 128).astype(f32)
    ap = (padP[:, 0:32:2, 0:32:2, :] + padP[:, 0:32:2, 1:32:2, :]
          + padP[:, 1:32:2, 0:32:2, :] + padP[:, 1:32:2, 1:32:2, :]) * 0.25
    ap = ap.astype(bf16)

    fill_bd(wbdA, g64[:, :, 128:192], 64, 64)
    b = conv3(padB, ap, lambda t: wbdA[t], 16, 128, 128, 2)
    fill_bd(wbdA, g64[:, :, 192:256], 64, 64)
    c = conv3(padB, b.reshape(2, 16, 16, 128), lambda t: wbdA[t], 16, 128, 128, 2,
              extra=ap.reshape(512, 128).astype(f32))

    fill_bd(wbd21, g64[:, :, 256:384], 64, 128)
    acc = conv_s2(padBs, c.reshape(2, 16, 16, 128), lambda t: wbd21[t],
                  16, 128, 256, 2)
    y1 = jnp.maximum(acc, 0.0).astype(bf16)
    sc = padBs[:, 1:17:2, 1:17:2, :].reshape(128, 128).astype(bf16)
    wscb2[0:64, 0:128] = wsc2[...]
    wscb2[64:128, 128:256] = wsc2[...]
    fill_bd(wbd22, g128[:, :, 0:128], 128, 128)
    y2 = conv3(padC, y1.reshape(2, 8, 8, 256), lambda t: wbd22[t], 8, 256, 256, 2,
               extra=jnp.dot(sc, wscb2[...], preferred_element_type=f32))

    v = y2.reshape(2, 8, 8, 256)
    y2s = jnp.concatenate([v[0:1, :, :, 0:128], v[0:1, :, :, 128:256],
                           v[1:2, :, :, 0:128], v[1:2, :, :, 128:256]], axis=0)

    acc = conv_s2(padCs, y2s, lambda t: g128[t, :, 128:384], 8, 128, 256, _B)
    y1 = jnp.maximum(acc, 0.0).astype(bf16)
    sc = padCs[:, 1:9:2, 1:9:2, :].reshape(_B * 16, 128).astype(bf16)
    y3 = conv3(padD3, y1.reshape(_B, 4, 4, 256), lambda t: g256[t, :, 0:256],
               4, 256, 256, _B,
               extra=jnp.dot(sc, wsc3[...], preferred_element_type=f32))

    padD[:, 1:5, 1:5, :] = y3.reshape(_B, 4, 4, 256)

    def pick22(di, dj):
        rows = jnp.concatenate([padD[:, di:di + 1, :, :],
                                padD[:, di + 2:di + 3, :, :]], axis=1)
        return jnp.concatenate([rows[:, :, dj:dj + 1, :],
                                rows[:, :, dj + 2:dj + 3, :]],
                               axis=2).reshape(_B * 4, 256)

    acc = jnp.zeros((_B * 4, 512), f32)
    for t, (di, dj) in enumerate([(i, j) for i in range(3) for j in range(3)]):
        acc = acc + jnp.dot(pick22(di, dj), g256[t, :, 256:768],
                            preferred_element_type=f32)
    y1 = jnp.maximum(acc, 0.0).astype(bf16)
    sc = pick22(1, 1)
    y4 = conv3(padE, y1.reshape(_B, 2, 2, 512), lambda t: w42[t], 2, 512, 512, _B,
               extra=jnp.dot(sc, wsc4[...], preferred_element_type=f32))

    y4r = y4.reshape(_B, 4, 512)
    lacc = jnp.zeros((_B, 128), f32)
    for p in range(4):
        lacc = lacc + jnp.dot(y4r[:, p, :], wfc[p], preferred_element_type=f32)
    out_ref[...] = lacc.reshape(1, _B, 128)


def _w9(w):
    return jnp.transpose(w, (2, 3, 1, 0)).reshape(9, w.shape[1], w.shape[0]).astype(jnp.bfloat16)


def _w1x1(w):
    return jnp.transpose(w[:, :, 0, 0]).astype(jnp.bfloat16)


def kernel(x, pre0, pre1, pre2, l1_conv1, l1_conv2, l2_conv1, l2_conv2, l2_sc,
           l3_conv1, l3_conv2, l3_sc, l4_conv1, l4_conv2, l4_sc, fc):
    nb = x.shape[0]
    xh = jnp.transpose(x, (0, 2, 3, 1)).astype(jnp.bfloat16)
    xp = jnp.pad(xh, ((0, 0), (1, 1), (1, 1), (0, 0))).reshape(2, 2, 2, 34, 34, 3)
    xp = jnp.transpose(xp, (0, 1, 3, 4, 2, 5)).reshape(2, 2, 34, 34, 6)

    nlab = fc.shape[0]
    fcr = jnp.transpose(fc.reshape(nlab, 512, 2, 2), (2, 3, 1, 0)).reshape(4, 512, nlab)
    fcr = jnp.pad(fcr, ((0, 0), (0, 0), (0, 128 - nlab))).astype(jnp.bfloat16)

    g64 = _w9(jnp.concatenate([pre1, pre2, l1_conv1, l1_conv2, l2_conv1],
                              axis=0))
    g128 = _w9(jnp.concatenate([l2_conv2, l3_conv1], axis=0))
    g256 = _w9(jnp.concatenate([l3_conv2, l4_conv1], axis=0))
    ws = [_w9(pre0), g64, g128, _w1x1(l2_sc), g256, _w1x1(l3_sc),
          _w9(l4_conv2), _w1x1(l4_sc), fcr]

    full = lambda arr: pl.BlockSpec(arr.shape, lambda i: (0,) * arr.ndim)
    in_specs = [pl.BlockSpec((1, 2, 34, 34, 6), lambda i: (i, 0, 0, 0, 0))]
    in_specs += [full(w) for w in ws]

    out = pl.pallas_call(
        _net_kernel,
        out_shape=jax.ShapeDtypeStruct((2, _B, 128), jnp.float32),
        grid=(2,),
        in_specs=in_specs,
        out_specs=pl.BlockSpec((1, _B, 128), lambda i: (i, 0, 0)),
        scratch_shapes=[
            pltpu.VMEM((2, 34, 32, 128), jnp.bfloat16),
            pltpu.VMEM((2, 32, 32, 128), jnp.float32),
            pltpu.VMEM((2, 18, 16, 128), jnp.bfloat16),
            pltpu.VMEM((2, 18, 18, 128), jnp.float32),
            pltpu.VMEM((2, 10, 8, 256), jnp.bfloat16),
            pltpu.VMEM((_B, 10, 10, 128), jnp.float32),
            pltpu.VMEM((_B, 6, 6, 256), jnp.bfloat16),
            pltpu.VMEM((_B, 6, 4, 256), jnp.bfloat16),
            pltpu.VMEM((_B, 4, 2, 512), jnp.bfloat16),
            pltpu.VMEM((9, 8, 128), jnp.bfloat16),
            pltpu.VMEM((9, 128, 128), jnp.bfloat16),
            pltpu.VMEM((9, 128, 256), jnp.bfloat16),
            pltpu.VMEM((9, 256, 256), jnp.bfloat16),
            pltpu.VMEM((128, 256), jnp.bfloat16),
        ],
        compiler_params=pltpu.CompilerParams(
            dimension_semantics=("parallel",),
            vmem_limit_bytes=_VMEM_LIMIT),
    )(xp, *ws)

    return out.reshape(nb, 128)[:, :nlab]


# speedup vs baseline: 1.0714x; 1.0714x over previous
import jax
import jax.numpy as jnp
from jax.experimental import pallas as pl
from jax.experimental.pallas import tpu as pltpu

_VMEM_LIMIT = 48 << 20
_B = 8


def _net_kernel(xp_ref, w0, g64, g128, wsc2, g256, wsc3, w42, wsc4, wfc, out_ref,
                padA, padP, padB, padBs, padC, padCs, padD, padD3, padE,
                wbd0, wbdA, wbd21, wbd22, wscb2):
    f32 = jnp.float32
    bf16 = jnp.bfloat16

    for p in (padA, padB, padBs, padC, padCs, padD, padD3, padE):
        p[...] = jnp.zeros(p.shape, p.dtype)
    for p in (wbd0, wbdA, wbd21, wbd22, wscb2):
        p[...] = jnp.zeros(p.shape, p.dtype)

    def fill_bd(scr, w, ci, co):
        scr[:, 0:ci, 0:co] = w
        scr[:, ci:2 * ci, co:2 * co] = w

    def conv3(pad, x, wslice, H, C, Co, B, extra=None, relu=True, lead=()):
        M = B * H * H
        acc = jnp.zeros((M, Co), f32)
        if x is not None:
            pad[:, 1:H + 1, :, :] = x
            zc = jnp.zeros((B, H + 2, 1, C), pad.dtype)
            full = pad[...]
            cols = [jnp.concatenate([zc, full[:, :, 0:H - 1, :]], axis=2),
                    full,
                    jnp.concatenate([full[:, :, 1:H, :], zc], axis=2)]
        else:
            cols = [pad[lead + (slice(None), slice(None),
                                slice(dj, dj + H), slice(None))]
                    for dj in range(3)]
        for dj in range(3):
            vdj = cols[dj]
            for di in range(3):
                xs = vdj[:, di:di + H, :, :].reshape(M, C)
                acc = acc + jnp.dot(xs, wslice(di * 3 + dj),
                                    preferred_element_type=f32)
        if extra is not None:
            acc = acc + extra
        if relu:
            acc = jnp.maximum(acc, 0.0)
        return acc.astype(bf16)

    def conv_s2(pad, x, wslice, H, C, Co, B):
        Ho = H // 2
        pad[:, 1:H + 1, 1:H + 1, :] = x.astype(f32)
        acc = jnp.zeros((B * Ho * Ho, Co), f32)
        for di in range(3):
            for dj in range(3):
                xs = pad[:, di:di + H:2, dj:dj + H:2, :].reshape(
                    B * Ho * Ho, C).astype(bf16)
                acc = acc + jnp.dot(xs, wslice(di * 3 + dj),
                                    preferred_element_type=f32)
        return acc

    fill_bd(wbd0, w0[...], 3, 64)
    a = conv3(xp_ref, None, lambda t: wbd0[t, 0:6, :], 32, 6, 128, 4)

    fill_bd(wbdA, g64[:, :, 0:64], 64, 64)
    a = conv3(padA, a.reshape(4, 32, 32, 128), lambda t: wbdA[t], 32, 128, 128, 4)
    fill_bd(wbdA, g64[:, :, 64:128], 64, 64)
    a = conv3(padA, a.reshape(4, 32, 32, 128), lambda t: wbdA[t], 32, 128, 128, 4)

    padP[...] = a.reshape(4, 32, 32, 128).astype(f32)
    ap = (padP[:, 0:32:2, 0:32:2, :] + padP[:, 0:32:2, 1:32:2, :]
          + padP[:, 1:32:2, 0:32:2, :] + padP[:, 1:32:2, 1:32:2, :]) * 0.25
    ap = ap.astype(bf16)

    fill_bd(wbdA, g64[:, :, 128:192], 64, 64)
    b = conv3(padB, ap, lambda t: wbdA[t], 16, 128, 128, 4)
    fill_bd(wbdA, g64[:, :, 192:256], 64, 64)
    c = conv3(padB, b.reshape(4, 16, 16, 128), lambda t: wbdA[t], 16, 128, 128, 4,
              extra=ap.reshape(1024, 128).astype(f32))

    fill_bd(wbd21, g64[:, :, 256:384], 64, 128)
    acc = conv_s2(padBs, c.reshape(4, 16, 16, 128), lambda t: wbd21[t],
                  16, 128, 256, 4)
    y1 = jnp.maximum(acc, 0.0).astype(bf16)
    sc = padBs[:, 1:17:2, 1:17:2, :].reshape(256, 128).astype(bf16)
    wscb2[0:64, 0:128] = wsc2[...]
    wscb2[64:128, 128:256] = wsc2[...]
    fill_bd(wbd22, g128[:, :, 0:128], 128, 128)
    y2 = conv3(padC, y1.reshape(4, 8, 8, 256), lambda t: wbd22[t], 8, 256, 256, 4,
               extra=jnp.dot(sc, wscb2[...], preferred_element_type=f32))

    v = y2.reshape(4, 8, 8, 256)
    y2s = jnp.concatenate(
        [v[p:p + 1, :, :, j * 128:(j + 1) * 128]
         for p in range(4) for j in range(2)], axis=0)

    acc = conv_s2(padCs, y2s, lambda t: g128[t, :, 128:384], 8, 128, 256, _B)
    y1 = jnp.maximum(acc, 0.0).astype(bf16)
    sc = padCs[:, 1:9:2, 1:9:2, :].reshape(_B * 16, 128).astype(bf16)
    y3 = conv3(padD3, y1.reshape(_B, 4, 4, 256), lambda t: g256[t, :, 0:256],
               4, 256, 256, _B,
               extra=jnp.dot(sc, wsc3[...], preferred_element_type=f32))

    padD[:, 1:5, 1:5, :] = y3.reshape(_B, 4, 4, 256)

    def pick22(di, dj):
        rows = jnp.concatenate([padD[:, di:di + 1, :, :],
                                padD[:, di + 2:di + 3, :, :]], axis=1)
        return jnp.concatenate([rows[:, :, dj:dj + 1, :],
                                rows[:, :, dj + 2:dj + 3, :]],
                               axis=2).reshape(_B * 4, 256)

    acc = jnp.zeros((_B * 4, 512), f32)
    for t, (di, dj) in enumerate([(i, j) for i in range(3) for j in range(3)]):
        acc = acc + jnp.dot(pick22(di, dj), g256[t, :, 256:768],
                            preferred_element_type=f32)
    y1 = jnp.maximum(acc, 0.0).astype(bf16)
    sc = pick22(1, 1)
    y4 = conv3(padE, y1.reshape(_B, 2, 2, 512), lambda t: w42[t], 2, 512, 512, _B,
               extra=jnp.dot(sc, wsc4[...], preferred_element_type=f32))

    y4r = y4.reshape(_B, 4, 512)
    lacc = jnp.zeros((_B, 128), f32)
    for p in range(4):
        lacc = lacc + jnp.dot(y4r[:, p, :], wfc[p], preferred_element_type=f32)
    out_ref[...] = lacc.reshape(1, _B, 128)


def _w9(w):
    return jnp.transpose(w, (2, 3, 1, 0)).reshape(9, w.shape[1], w.shape[0]).astype(jnp.bfloat16)


def _w1x1(w):
    return jnp.transpose(w[:, :, 0, 0]).astype(jnp.bfloat16)


def kernel(x, pre0, pre1, pre2, l1_conv1, l1_conv2, l2_conv1, l2_conv2, l2_sc,
           l3_conv1, l3_conv2, l3_sc, l4_conv1, l4_conv2, l4_sc, fc):
    nb = x.shape[0]
    xh = jnp.transpose(x, (0, 2, 3, 1)).astype(jnp.bfloat16)
    xp = jnp.pad(xh, ((0, 0), (1, 1), (1, 1), (0, 0))).reshape(4, 2, 34, 34, 3)
    xp = jnp.transpose(xp, (0, 2, 3, 1, 4)).reshape(4, 34, 34, 6)

    nlab = fc.shape[0]
    fcr = jnp.transpose(fc.reshape(nlab, 512, 2, 2), (2, 3, 1, 0)).reshape(4, 512, nlab)
    fcr = jnp.pad(fcr, ((0, 0), (0, 0), (0, 128 - nlab))).astype(jnp.bfloat16)

    g64 = _w9(jnp.concatenate([pre1, pre2, l1_conv1, l1_conv2, l2_conv1],
                              axis=0))
    g128 = _w9(jnp.concatenate([l2_conv2, l3_conv1], axis=0))
    g256 = _w9(jnp.concatenate([l3_conv2, l4_conv1], axis=0))
    ws = [_w9(pre0), g64, g128, _w1x1(l2_sc), g256, _w1x1(l3_sc),
          _w9(l4_conv2), _w1x1(l4_sc), fcr]

    full = lambda arr: pl.BlockSpec(arr.shape, lambda i: (0,) * arr.ndim)
    in_specs = [full(xp)] + [full(w) for w in ws]

    out = pl.pallas_call(
        _net_kernel,
        out_shape=jax.ShapeDtypeStruct((1, _B, 128), jnp.float32),
        grid=(1,),
        in_specs=in_specs,
        out_specs=pl.BlockSpec((1, _B, 128), lambda i: (i, 0, 0)),
        scratch_shapes=[
            pltpu.VMEM((4, 34, 32, 128), jnp.bfloat16),
            pltpu.VMEM((4, 32, 32, 128), jnp.float32),
            pltpu.VMEM((4, 18, 16, 128), jnp.bfloat16),
            pltpu.VMEM((4, 18, 18, 128), jnp.float32),
            pltpu.VMEM((4, 10, 8, 256), jnp.bfloat16),
            pltpu.VMEM((_B, 10, 10, 128), jnp.float32),
            pltpu.VMEM((_B, 6, 6, 256), jnp.bfloat16),
            pltpu.VMEM((_B, 6, 4, 256), jnp.bfloat16),
            pltpu.VMEM((_B, 4, 2, 512), jnp.bfloat16),
            pltpu.VMEM((9, 8, 128), jnp.bfloat16),
            pltpu.VMEM((9, 128, 128), jnp.bfloat16),
            pltpu.VMEM((9, 128, 256), jnp.bfloat16),
            pltpu.VMEM((9, 256, 256), jnp.bfloat16),
            pltpu.VMEM((128, 256), jnp.bfloat16),
        ],
        compiler_params=pltpu.CompilerParams(
            dimension_semantics=("parallel",),
            vmem_limit_bytes=_VMEM_LIMIT),
    )(xp, *ws)

    return out.reshape(nb, 128)[:, :nlab]
